# baseline (device time: 52750 ns/iter reference)
import jax
import jax.numpy as jnp
from jax import lax
from jax.experimental import pallas as pl
from jax.experimental.pallas import tpu as pltpu

N_Y = 2
QROWS = 1024
CROWS = 128
KQ = QROWS // CROWS
DY_LO, DY_SZ = 0, 336
DZ_LO, DZ_SZ = 336, 352
DX_LO, DX_SZ = 688, 336
DZ_FEED = (2, 3, 4, 5)
DX_FEED = (5, 6, 7)
NY = KQ + 1
NZ = KQ + 1
NX = KQ + 1

BF16 = jnp.bfloat16


def kernel(x):
    m_per, n = x.shape
    m_out = N_Y * m_per
    assert m_per == 4 * QROWS

    def body(x_ref, out_ref, send_y, recv_y, send_z, recv_z, send_x, recv_x):
        my_x = lax.axis_index("x")
        my_y = lax.axis_index("y")
        my_z = lax.axis_index("z")
        p = my_z % 2
        c = 2 * my_x + p
        d = 3 - c
        qz = 2 * my_x + (1 - p)
        qx = 2 * (1 - my_x) + p
        nbr_y = (my_x, 1 - my_y, my_z)
        nbr_z = (my_x, my_y, my_z + 1 - 2 * p)
        nbr_x = (1 - my_x, my_y, my_z)

        barrier_sem = pltpu.get_barrier_semaphore()
        for dev in (nbr_y, nbr_z, nbr_x):
            pl.semaphore_signal(
                barrier_sem, inc=1, device_id=dev,
                device_id_type=pl.DeviceIdType.MESH,
            )
        pl.semaphore_wait(barrier_sem, 3)

        my_base = my_y * m_per
        ob = (1 - my_y) * m_per

        def copy(src, dst, sems, slot, dev):
            send_sem, recv_sem = sems
            return pltpu.make_async_remote_copy(
                src_ref=src, dst_ref=dst,
                send_sem=send_sem.at[slot], recv_sem=recv_sem.at[slot],
                device_id=dev, device_id_type=pl.DeviceIdType.MESH,
            )

        def to_bf16(rows_off, rows):
            out_ref[pl.ds(my_base + rows_off, rows), :] = x_ref[
                pl.ds(rows_off, rows), :
            ].astype(BF16)

        y_rdmas = []
        for k in range(KQ):
            off = c * QROWS + k * CROWS
            to_bf16(off, CROWS)
            r = copy(
                out_ref.at[pl.ds(my_base + off, CROWS), :],
                out_ref.at[pl.ds(my_base + off, CROWS), :],
                (send_y, recv_y), k, nbr_y,
            )
            r.start()
            y_rdmas.append(r)
        dy_off = d * QROWS + DY_LO
        to_bf16(dy_off, DY_SZ)
        r = copy(
            out_ref.at[pl.ds(my_base + dy_off, DY_SZ), :],
            out_ref.at[pl.ds(my_base + dy_off, DY_SZ), :],
            (send_y, recv_y), KQ, nbr_y,
        )
        r.start()
        y_rdmas.append(r)

        rest_pieces = [
            (qz * QROWS, 512),
            (qz * QROWS + 512, 512),
            (qx * QROWS, 512),
            (qx * QROWS + 512, 512),
            (d * QROWS + DY_SZ, 352),
            (d * QROWS + DY_SZ + 352, 336),
        ]

        z_rdmas = [None] * NZ
        x_rdmas = [None] * NX
        for k in range(KQ):
            off = ob + c * QROWS + k * CROWS
            y_rdmas[k].wait_recv()
            rz = copy(
                out_ref.at[pl.ds(off, CROWS), :],
                out_ref.at[pl.ds(off, CROWS), :],
                (send_z, recv_z), k, nbr_z,
            )
            rz.start()
            z_rdmas[k] = rz
            rx = copy(
                out_ref.at[pl.ds(off, CROWS), :],
                out_ref.at[pl.ds(off, CROWS), :],
                (send_x, recv_x), k, nbr_x,
            )
            rx.start()
            x_rdmas[k] = rx
            if k < len(rest_pieces):
                to_bf16(*rest_pieces[k])

        for k in DZ_FEED:
            x_rdmas[k].wait_recv()
        dz_off = ob + qx * QROWS + DZ_LO
        r = copy(
            out_ref.at[pl.ds(dz_off, DZ_SZ), :],
            out_ref.at[pl.ds(dz_off, DZ_SZ), :],
            (send_z, recv_z), KQ, nbr_z,
        )
        r.start()
        z_rdmas[KQ] = r

        for k in DX_FEED:
            z_rdmas[k].wait_recv()
        dx_off = ob + qz * QROWS + DX_LO
        r = copy(
            out_ref.at[pl.ds(dx_off, DX_SZ), :],
            out_ref.at[pl.ds(dx_off, DX_SZ), :],
            (send_x, recv_x), KQ, nbr_x,
        )
        r.start()
        x_rdmas[KQ] = r

        for r in y_rdmas:
            r.wait_send()
        y_rdmas[KQ].wait_recv()
        for i, r in enumerate(z_rdmas):
            r.wait_send()
            if i not in DX_FEED:
                r.wait_recv()
        for i, r in enumerate(x_rdmas):
            r.wait_send()
            if i not in DZ_FEED:
                r.wait_recv()

    return pl.pallas_call(
        body,
        out_shape=jax.ShapeDtypeStruct((m_out, n), BF16),
        in_specs=[pl.BlockSpec(memory_space=pltpu.VMEM)],
        out_specs=pl.BlockSpec(memory_space=pltpu.VMEM),
        scratch_shapes=[
            pltpu.SemaphoreType.DMA((NY,)),
            pltpu.SemaphoreType.DMA((NY,)),
            pltpu.SemaphoreType.DMA((NZ,)),
            pltpu.SemaphoreType.DMA((NZ,)),
            pltpu.SemaphoreType.DMA((NX,)),
            pltpu.SemaphoreType.DMA((NX,)),
        ],
        compiler_params=pltpu.CompilerParams(collective_id=0),
    )(x)


# device time: 14858 ns/iter; 3.5503x vs baseline; 3.5503x over previous
import jax
import jax.numpy as jnp
from jax import lax
from jax.experimental import pallas as pl
from jax.experimental.pallas import tpu as pltpu

N_Y = 2
QROWS = 1024
CROWS = 128
KQ = QROWS // CROWS
DY_LO, DY_SZ = 0, 576
DZ_LO, DZ_SZ = 576, 224
DX_LO, DX_SZ = 800, 224
DZ_FEED = (4, 5, 6)
DX_FEED = (6, 7)
NY = KQ + 1
NZ = KQ + 1
NX = KQ + 1

BF16 = jnp.bfloat16


def kernel(x):
    m_per, n = x.shape
    m_out = N_Y * m_per
    assert m_per == 4 * QROWS

    def body(x_ref, out_ref, send_y, recv_y, send_z, recv_z, send_x, recv_x):
        my_x = lax.axis_index("x")
        my_y = lax.axis_index("y")
        my_z = lax.axis_index("z")
        p = my_z % 2
        c = 2 * my_x + p
        d = 3 - c
        qz = 2 * my_x + (1 - p)
        qx = 2 * (1 - my_x) + p
        nbr_y = (my_x, 1 - my_y, my_z)
        nbr_z = (my_x, my_y, my_z + 1 - 2 * p)
        nbr_x = (1 - my_x, my_y, my_z)

        barrier_sem = pltpu.get_barrier_semaphore()
        for dev in (nbr_y, nbr_z, nbr_x):
            pl.semaphore_signal(
                barrier_sem, inc=1, device_id=dev,
                device_id_type=pl.DeviceIdType.MESH,
            )
        pl.semaphore_wait(barrier_sem, 3)

        my_base = my_y * m_per
        ob = (1 - my_y) * m_per

        def copy(src, dst, sems, slot, dev):
            send_sem, recv_sem = sems
            return pltpu.make_async_remote_copy(
                src_ref=src, dst_ref=dst,
                send_sem=send_sem.at[slot], recv_sem=recv_sem.at[slot],
                device_id=dev, device_id_type=pl.DeviceIdType.MESH,
            )

        def to_bf16(rows_off, rows):
            out_ref[pl.ds(my_base + rows_off, rows), :] = x_ref[
                pl.ds(rows_off, rows), :
            ].astype(BF16)

        y_rdmas = []
        for k in range(KQ):
            off = c * QROWS + k * CROWS
            to_bf16(off, CROWS)
            r = copy(
                out_ref.at[pl.ds(my_base + off, CROWS), :],
                out_ref.at[pl.ds(my_base + off, CROWS), :],
                (send_y, recv_y), k, nbr_y,
            )
            r.start()
            y_rdmas.append(r)
        dy_off = d * QROWS + DY_LO
        to_bf16(dy_off, DY_SZ)
        r = copy(
            out_ref.at[pl.ds(my_base + dy_off, DY_SZ), :],
            out_ref.at[pl.ds(my_base + dy_off, DY_SZ), :],
            (send_y, recv_y), KQ, nbr_y,
        )
        r.start()
        y_rdmas.append(r)

        rest_pieces = [
            (qz * QROWS, 512),
            (qz * QROWS + 512, 512),
            (qx * QROWS, 512),
            (qx * QROWS + 512, 512),
            (d * QROWS + DY_SZ, 224),
            (d * QROWS + DY_SZ + 224, 224),
        ]

        z_rdmas = [None] * NZ
        x_rdmas = [None] * NX
        for k in range(KQ):
            off = ob + c * QROWS + k * CROWS
            y_rdmas[k].wait_recv()
            rz = copy(
                out_ref.at[pl.ds(off, CROWS), :],
                out_ref.at[pl.ds(off, CROWS), :],
                (send_z, recv_z), k, nbr_z,
            )
            rz.start()
            z_rdmas[k] = rz
            rx = copy(
                out_ref.at[pl.ds(off, CROWS), :],
                out_ref.at[pl.ds(off, CROWS), :],
                (send_x, recv_x), k, nbr_x,
            )
            rx.start()
            x_rdmas[k] = rx
            if k < len(rest_pieces):
                to_bf16(*rest_pieces[k])

        for k in DZ_FEED:
            x_rdmas[k].wait_recv()
        dz_off = ob + qx * QROWS + DZ_LO
        r = copy(
            out_ref.at[pl.ds(dz_off, DZ_SZ), :],
            out_ref.at[pl.ds(dz_off, DZ_SZ), :],
            (send_z, recv_z), KQ, nbr_z,
        )
        r.start()
        z_rdmas[KQ] = r

        for k in DX_FEED:
            z_rdmas[k].wait_recv()
        dx_off = ob + qz * QROWS + DX_LO
        r = copy(
            out_ref.at[pl.ds(dx_off, DX_SZ), :],
            out_ref.at[pl.ds(dx_off, DX_SZ), :],
            (send_x, recv_x), KQ, nbr_x,
        )
        r.start()
        x_rdmas[KQ] = r

        for r in y_rdmas:
            r.wait_send()
        y_rdmas[KQ].wait_recv()
        for i, r in enumerate(z_rdmas):
            r.wait_send()
            if i not in DX_FEED:
                r.wait_recv()
        for i, r in enumerate(x_rdmas):
            r.wait_send()
            if i not in DZ_FEED:
                r.wait_recv()

    return pl.pallas_call(
        body,
        out_shape=jax.ShapeDtypeStruct((m_out, n), BF16),
        in_specs=[pl.BlockSpec(memory_space=pltpu.VMEM)],
        out_specs=pl.BlockSpec(memory_space=pltpu.VMEM),
        scratch_shapes=[
            pltpu.SemaphoreType.DMA((NY,)),
            pltpu.SemaphoreType.DMA((NY,)),
            pltpu.SemaphoreType.DMA((NZ,)),
            pltpu.SemaphoreType.DMA((NZ,)),
            pltpu.SemaphoreType.DMA((NX,)),
            pltpu.SemaphoreType.DMA((NX,)),
        ],
        compiler_params=pltpu.CompilerParams(collective_id=0),
    )(x)
